# Initial kernel scaffold; baseline (speedup 1.0000x reference)
import jax
import jax.numpy as jnp
from jax.experimental import pallas as pl


def kernel(xyz, features, W_ft, b_ft, ln_ft_g, ln_ft_b, conv1_w, conv1_b, bn_g, bn_b, conv2_w, conv2_b, W_fu, b_fu, ln_fu_g, ln_fu_b, alpha, beta):
    raise NotImplementedError("write your pallas kernel here")


# speedup vs baseline: 13.6050x; 13.6050x over previous
import functools

import jax
import jax.numpy as jnp
from jax import lax
from jax.experimental import pallas as pl
from jax.experimental.pallas import tpu as pltpu
from jax.experimental.pallas import tpu_sc as plsc

B, N, K = 8, 2048, 16
DIM = 128
HID = 64
RBLK = 256
PBLK = 512
GBLK = RBLK * K
CNT = float(B * N * K)


def _point_transform_body(f_ref, wft_ref, bft_ref, g_ref, b_ref, wfu_ref, u_ref):
    f = f_ref[...]
    t = lax.dot_general(f, wft_ref[...], (((1,), (1,)), ((), ()))) + bft_ref[...]
    m = jnp.mean(t, axis=1, keepdims=True)
    v = jnp.mean((t - m) ** 2, axis=1, keepdims=True)
    t = (t - m) / jnp.sqrt(v + 1e-5) * g_ref[...] + b_ref[...]
    t = t * jax.nn.sigmoid(t)
    w1 = wfu_ref[:, :DIM]
    u_ref[...] = lax.dot_general(t, w1, (((1,), (1,)), ((), ())))


def _point_transform(feat_flat, W_ft, b_ft2, g2, b2, W_fu):
    nrows = feat_flat.shape[0]
    return pl.pallas_call(
        _point_transform_body,
        grid=(nrows // PBLK,),
        in_specs=[
            pl.BlockSpec((PBLK, DIM), lambda i: (i, 0)),
            pl.BlockSpec((DIM, DIM), lambda i: (0, 0)),
            pl.BlockSpec((1, DIM), lambda i: (0, 0)),
            pl.BlockSpec((1, DIM), lambda i: (0, 0)),
            pl.BlockSpec((1, DIM), lambda i: (0, 0)),
            pl.BlockSpec((DIM, 2 * DIM), lambda i: (0, 0)),
        ],
        out_specs=pl.BlockSpec((PBLK, DIM), lambda i: (i, 0)),
        out_shape=jax.ShapeDtypeStruct((nrows, DIM), jnp.float32),
    )(feat_flat, W_ft, b_ft2, g2, b2, W_fu)


def _knn_body(xyz_ref, xyzt_ref, idx_ref):
    xb = xyz_ref[...]
    xt = xyzt_ref[...]
    dot = jnp.dot(xb.astype(jnp.bfloat16), xt.astype(jnp.bfloat16),
                  preferred_element_type=jnp.float32)
    sq_r = (xb[:, 0:1] ** 2 + xb[:, 1:2] ** 2) + xb[:, 2:3] ** 2
    sq_c = (xt[0:1, :] ** 2 + xt[1:2, :] ** 2) + xt[2:3, :] ** 2
    d = (-2.0 * dot + sq_r) + sq_c
    iota = lax.broadcasted_iota(jnp.int32, (1, N), 1)
    inf = jnp.float32(jnp.inf)
    cols = []
    for _ in range(K):
        m = jnp.min(d, axis=1, keepdims=True)
        cand = jnp.where(d == m, iota, N)
        sel = jnp.min(cand, axis=1, keepdims=True)
        cols.append(sel)
        d = jnp.where(iota == sel, inf, d)
    idx_ref[...] = jnp.concatenate(cols, axis=1)


def _knn1(xyz_b, xyzT_b):
    return pl.pallas_call(
        _knn_body,
        grid=(N // RBLK,),
        in_specs=[
            pl.BlockSpec((RBLK, 3), lambda r: (r, 0)),
            pl.BlockSpec((3, N), lambda r: (0, 0)),
        ],
        out_specs=pl.BlockSpec((RBLK, K), lambda r: (r, 0)),
        out_shape=jax.ShapeDtypeStruct((N, K), jnp.int32),
    )(xyz_b, xyzT_b)


def _sc_gather(tab_u, tab_x, idx3):
    info = plsc.get_sparse_core_info()
    nc, ns = info.num_cores, info.num_subcores
    nw = nc * ns
    n_ch, ch = idx3.shape[1], idx3.shape[2]
    per_w = n_ch * ch
    total = nw * per_w
    mesh = plsc.VectorSubcoreMesh(core_axis_name="c", subcore_axis_name="s")

    @functools.partial(
        pl.kernel,
        mesh=mesh,
        out_type=[
            jax.ShapeDtypeStruct((total, DIM), jnp.float32),
            jax.ShapeDtypeStruct((total, DIM), jnp.float32),
        ],
        scratch_types=[
            pltpu.VMEM((n_ch, ch), jnp.int32),
            pltpu.VMEM((ch, DIM), jnp.float32),
            pltpu.VMEM((ch, DIM), jnp.float32),
            pltpu.SemaphoreType.DMA,
            pltpu.SemaphoreType.DMA,
        ],
    )
    def gather_kernel(tab_u_hbm, tab_x_hbm, idx_hbm, out_u_hbm, out_x_hbm,
                      idx_v, buf_u, buf_x, sem_u, sem_x):
        wid = lax.axis_index("s") * nc + lax.axis_index("c")
        pltpu.sync_copy(idx_hbm.at[wid], idx_v)

        def body(j, carry):
            cp_u = pltpu.async_copy(tab_u_hbm.at[idx_v.at[j]], buf_u, sem_u)
            cp_x = pltpu.async_copy(tab_x_hbm.at[idx_v.at[j]], buf_x, sem_x)
            cp_u.wait()
            cp_x.wait()
            base = wid * per_w + j * ch
            pltpu.sync_copy(buf_u, out_u_hbm.at[pl.ds(base, ch)])
            pltpu.sync_copy(buf_x, out_x_hbm.at[pl.ds(base, ch)])
            return carry

        lax.fori_loop(0, n_ch, body, 0)

    return gather_kernel(tab_u, tab_x, idx3)


def _bn_stats_body(nxyz_ref, ctr_ref, w1_ref, b1_ref, out_ref):
    rel3 = nxyz_ref[...].reshape(RBLK, K, DIM) - ctr_ref[...].reshape(RBLK, 1, DIM)
    rel = rel3.reshape(GBLK, DIM)[:, :3]
    h = lax.dot_general(rel, w1_ref[...], (((1,), (1,)), ((), ()))) + b1_ref[...]

    @pl.when(pl.program_id(0) == 0)
    def _():
        out_ref[...] = jnp.zeros_like(out_ref)

    s1 = jnp.sum(h, axis=0, keepdims=True)
    s2 = jnp.sum(h * h, axis=0, keepdims=True)
    out_ref[...] += jnp.concatenate([s1, s2], axis=0)


def _bn_stats(x_g, ctr, conv1_w, conv1_b2):
    nrows = ctr.shape[0]
    return pl.pallas_call(
        _bn_stats_body,
        grid=(nrows // RBLK,),
        in_specs=[
            pl.BlockSpec((GBLK, DIM), lambda i: (i, 0)),
            pl.BlockSpec((RBLK, DIM), lambda i: (i, 0)),
            pl.BlockSpec((HID, 3), lambda i: (0, 0)),
            pl.BlockSpec((1, HID), lambda i: (0, 0)),
        ],
        out_specs=pl.BlockSpec((2, HID), lambda i: (0, 0)),
        out_shape=jax.ShapeDtypeStruct((2, HID), jnp.float32),
    )(x_g, ctr, conv1_w, conv1_b2)


def _fused_body(ug_ref, nxyz_ref, ctr_ref, stats_ref, w1_ref, b1_ref,
                bng_ref, bnb_ref, wfu_ref, w2_ref, b2_ref, bfu_ref,
                lng_ref, lnb_ref, al_ref, be_ref, out_ref):
    rel3 = nxyz_ref[...].reshape(RBLK, K, DIM) - ctr_ref[...].reshape(RBLK, 1, DIM)
    rel = rel3.reshape(GBLK, DIM)[:, :3]
    h = lax.dot_general(rel, w1_ref[...], (((1,), (1,)), ((), ()))) + b1_ref[...]
    mean = stats_ref[0:1, :] / CNT
    var = stats_ref[1:2, :] / CNT - mean * mean
    h = (h - mean) / jnp.sqrt(var + 1e-5) * bng_ref[...] + bnb_ref[...]
    h = h * jax.nn.sigmoid(h)
    m2 = jnp.dot(wfu_ref[:, DIM:], w2_ref[...])
    b_eff = lax.dot_general(b2_ref[...], wfu_ref[:, DIM:],
                            (((1,), (1,)), ((), ())))
    pre = ug_ref[...] + lax.dot_general(h, m2, (((1,), (1,)), ((), ())))
    pre = pre + b_eff + bfu_ref[...]
    mu = jnp.mean(pre, axis=1, keepdims=True)
    v = jnp.mean((pre - mu) ** 2, axis=1, keepdims=True)
    f = (pre - mu) / jnp.sqrt(v + 1e-5) * lng_ref[...] + lnb_ref[...]
    f = f * jax.nn.sigmoid(f)
    f = al_ref[...] * f + be_ref[...]
    f3 = f.reshape(RBLK, K, DIM)
    s = jnp.sum(f3, axis=2)
    e = jnp.exp(s - jnp.max(s, axis=1, keepdims=True))
    w = e / jnp.sum(e, axis=1, keepdims=True)
    out_ref[...] = jnp.sum(f3 * w[:, :, None], axis=1)


def _fused(u_g, x_g, ctr, stats, conv1_w, conv1_b2, bn_g2, bn_b2,
           W_fu, conv2_w, conv2_b2, b_fu2, ln_g2, ln_b2, al2, be2):
    nrows = ctr.shape[0]
    return pl.pallas_call(
        _fused_body,
        grid=(nrows // RBLK,),
        in_specs=[
            pl.BlockSpec((GBLK, DIM), lambda i: (i, 0)),
            pl.BlockSpec((GBLK, DIM), lambda i: (i, 0)),
            pl.BlockSpec((RBLK, DIM), lambda i: (i, 0)),
            pl.BlockSpec((2, HID), lambda i: (0, 0)),
            pl.BlockSpec((HID, 3), lambda i: (0, 0)),
            pl.BlockSpec((1, HID), lambda i: (0, 0)),
            pl.BlockSpec((1, HID), lambda i: (0, 0)),
            pl.BlockSpec((1, HID), lambda i: (0, 0)),
            pl.BlockSpec((DIM, 2 * DIM), lambda i: (0, 0)),
            pl.BlockSpec((DIM, HID), lambda i: (0, 0)),
            pl.BlockSpec((1, DIM), lambda i: (0, 0)),
            pl.BlockSpec((1, DIM), lambda i: (0, 0)),
            pl.BlockSpec((1, DIM), lambda i: (0, 0)),
            pl.BlockSpec((1, DIM), lambda i: (0, 0)),
            pl.BlockSpec((1, DIM), lambda i: (0, 0)),
            pl.BlockSpec((1, DIM), lambda i: (0, 0)),
        ],
        out_specs=pl.BlockSpec((RBLK, DIM), lambda i: (i, 0)),
        out_shape=jax.ShapeDtypeStruct((nrows, DIM), jnp.float32),
    )(u_g, x_g, ctr, stats, conv1_w, conv1_b2, bn_g2, bn_b2,
      W_fu, conv2_w, conv2_b2, b_fu2, ln_g2, ln_b2, al2, be2)


def kernel(xyz, features, W_ft, b_ft, ln_ft_g, ln_ft_b, conv1_w, conv1_b,
           bn_g, bn_b, conv2_w, conv2_b, W_fu, b_fu, ln_fu_g, ln_fu_b,
           alpha, beta):
    feat_flat = features.reshape(B * N, DIM)
    xyz_flat = xyz.reshape(B * N, 3)
    tab_x = jnp.pad(xyz_flat, ((0, 0), (0, DIM - 3)))
    xyzT = xyz.transpose(0, 2, 1)

    row = lambda a: a.reshape(1, -1)
    tab_u = _point_transform(feat_flat, W_ft, row(b_ft), row(ln_ft_g),
                             row(ln_ft_b), W_fu)
    ug_list, xg_list, st_list = [], [], []
    for b in range(B):
        knn_b = _knn1(xyz[b], xyzT[b])
        idx3 = knn_b.reshape(32, -1, 128)
        sl = slice(b * N, (b + 1) * N)
        u_g, x_g = _sc_gather(tab_u[sl], tab_x[sl], idx3)
        ug_list.append(u_g)
        xg_list.append(x_g)
        st_list.append(_bn_stats(x_g, tab_x[sl], conv1_w, row(conv1_b)))
    stats = st_list[0]
    for s in st_list[1:]:
        stats = stats + s
    outs = []
    for b in range(B):
        sl = slice(b * N, (b + 1) * N)
        outs.append(_fused(
            ug_list[b], xg_list[b], tab_x[sl], stats,
            conv1_w, row(conv1_b), row(bn_g), row(bn_b), W_fu, conv2_w,
            row(conv2_b), row(b_fu), row(ln_fu_g), row(ln_fu_b),
            alpha.reshape(1, DIM), beta.reshape(1, DIM)))
    return jnp.stack(outs).reshape(B, N, DIM)
